# Initial kernel scaffold; baseline (speedup 1.0000x reference)
import jax
import jax.numpy as jnp
from jax.experimental import pallas as pl


def kernel(trajectory, occupancy, voxel_coords):
    raise NotImplementedError("write your pallas kernel here")


# speedup vs baseline: 11.3092x; 11.3092x over previous
import functools
import math

import jax
import jax.numpy as jnp
from jax import lax
from jax.experimental import pallas as pl
from jax.experimental.pallas import tpu as pltpu
from jax.experimental.pallas import tpu_sc as plsc

_EGO_LENGTH = 4.7
_EGO_WIDTH = 1.85
_SAFETY_MARGIN = 0.5
_HALF_DIAG = math.sqrt(
    (_EGO_LENGTH / 2 + _SAFETY_MARGIN) ** 2 + (_EGO_WIDTH / 2 + _SAFETY_MARGIN) ** 2
)

_L = 16
_NC = 2
_NS = 16
_NW = _NC * _NS
_T = 64
_QG = 8
_NG = _T // _QG
_SENTINEL = 1.0e18


def _sc_partial_min(xs, ys, occ, pxb, pyb):
    n = xs.shape[0]
    p_per_w = n // _NW
    nblk = p_per_w // _L
    mesh = plsc.VectorSubcoreMesh(core_axis_name="c", subcore_axis_name="s")

    @functools.partial(
        pl.kernel,
        out_type=jax.ShapeDtypeStruct((_NW, _T, _L), jnp.float32),
        mesh=mesh,
        scratch_types=[
            pltpu.VMEM((p_per_w + _L,), jnp.float32),
            pltpu.VMEM((p_per_w + _L,), jnp.float32),
            pltpu.VMEM((p_per_w,), jnp.float32),
            pltpu.VMEM((_T * _L,), jnp.float32),
            pltpu.VMEM((_T * _L,), jnp.float32),
            pltpu.VMEM((_T, _L), jnp.float32),
        ],
    )
    def sc_kernel(xs_hbm, ys_hbm, occ_hbm, pxb_hbm, pyb_hbm, out_hbm,
                  x_v, y_v, o_v, a_v, b_v, acc_v):
        wid = lax.axis_index("c") * _NS + lax.axis_index("s")
        base = wid * p_per_w
        pltpu.sync_copy(xs_hbm.at[pl.ds(base, p_per_w)], x_v.at[pl.ds(0, p_per_w)])
        pltpu.sync_copy(ys_hbm.at[pl.ds(base, p_per_w)], y_v.at[pl.ds(0, p_per_w)])
        pltpu.sync_copy(occ_hbm.at[pl.ds(base, p_per_w)], o_v)
        pltpu.sync_copy(pxb_hbm, a_v)
        pltpu.sync_copy(pyb_hbm, b_v)

        sent = jnp.full((_L,), _SENTINEL, jnp.float32)

        def cbody(i, carry):
            off = i * _L
            xv = x_v[pl.ds(off, _L)]
            ov = o_v[pl.ds(off, _L)]
            x_v[pl.ds(off, _L)] = jnp.where(ov > 0.5, xv, sent)
            return carry

        lax.fori_loop(0, nblk, cbody, jnp.int32(0))
        nblk_c = nblk

        inf16 = jnp.full((_L,), jnp.inf, jnp.float32)
        for g in range(_NG):
            pa = [a_v[pl.ds((g * _QG + j) * _L, _L)] for j in range(_QG)]
            pb = [b_v[pl.ds((g * _QG + j) * _L, _L)] for j in range(_QG)]

            def sbody(i, accs, pa=pa, pb=pb):
                off = i * _L
                xv = x_v[pl.ds(off, _L)]
                yv = y_v[pl.ds(off, _L)]
                out = []
                for j, acc in enumerate(accs):
                    dx = xv - pa[j]
                    dy = yv - pb[j]
                    out.append(jnp.minimum(acc, dx * dx + dy * dy))
                return tuple(out)

            accs = lax.fori_loop(0, nblk_c, sbody, (inf16,) * _QG)
            for j in range(_QG):
                acc_v[g * _QG + j, :] = accs[j]
        pltpu.sync_copy(acc_v, out_hbm.at[wid])

    return sc_kernel(xs, ys, occ, pxb, pyb)


def _tc_combine(partials_2d):
    def body(p_ref, cf_ref, md_ref):
        d2 = jnp.min(p_ref[...], axis=0, keepdims=True)
        md = jnp.sqrt(d2)
        md_ref[...] = md
        cf_ref[...] = md >= _HALF_DIAG

    return pl.pallas_call(
        body,
        out_shape=(
            jax.ShapeDtypeStruct((1, _T), jnp.bool_),
            jax.ShapeDtypeStruct((1, _T), jnp.float32),
        ),
    )(partials_2d)


def kernel(trajectory, occupancy, voxel_coords):
    n = occupancy.size
    xs = voxel_coords[..., 0].reshape(n)
    ys = voxel_coords[..., 1].reshape(n)
    occ = occupancy.reshape(n)

    px = trajectory[:, 0].astype(jnp.float32)
    py = trajectory[:, 1].astype(jnp.float32)
    pxb = jnp.broadcast_to(px[:, None], (_T, _L)).reshape(_T * _L)
    pyb = jnp.broadcast_to(py[:, None], (_T, _L)).reshape(_T * _L)

    partials = _sc_partial_min(xs, ys, occ, pxb, pyb)
    partials_2d = partials.transpose(0, 2, 1).reshape(_NW * _L, _T)
    cf, md = _tc_combine(partials_2d)
    return cf.reshape(_T), md.reshape(_T)


# speedup vs baseline: 16.1160x; 1.4250x over previous
import functools
import math

import jax
import jax.numpy as jnp
from jax import lax
from jax.experimental import pallas as pl
from jax.experimental.pallas import tpu as pltpu
from jax.experimental.pallas import tpu_sc as plsc

_EGO_LENGTH = 4.7
_EGO_WIDTH = 1.85
_SAFETY_MARGIN = 0.5
_HALF_DIAG = math.sqrt(
    (_EGO_LENGTH / 2 + _SAFETY_MARGIN) ** 2 + (_EGO_WIDTH / 2 + _SAFETY_MARGIN) ** 2
)

_L = 16
_NC = 2
_NS = 16
_NW = _NC * _NS
_T = 64
_QG = 8
_NG = _T // _QG
_SENTINEL = 1.0e18


def _sc_partial_min(xs, ys, occ, pxb, pyb):
    n = xs.shape[0]
    p_per_w = n // _NW
    nblk = p_per_w // _L
    mesh = plsc.VectorSubcoreMesh(core_axis_name="c", subcore_axis_name="s")

    @functools.partial(
        pl.kernel,
        out_type=jax.ShapeDtypeStruct((_NW, _T, _L), jnp.float32),
        mesh=mesh,
        compiler_params=pltpu.CompilerParams(needs_layout_passes=False),
        scratch_types=[
            pltpu.VMEM((p_per_w + _L,), jnp.float32),
            pltpu.VMEM((p_per_w + _L,), jnp.float32),
            pltpu.VMEM((p_per_w,), jnp.float32),
            pltpu.VMEM((_T * _L,), jnp.float32),
            pltpu.VMEM((_T * _L,), jnp.float32),
            pltpu.VMEM((_T, _L), jnp.float32),
        ],
    )
    def sc_kernel(xs_hbm, ys_hbm, occ_hbm, pxb_hbm, pyb_hbm, out_hbm,
                  x_v, y_v, o_v, a_v, b_v, acc_v):
        wid = lax.axis_index("c") * _NS + lax.axis_index("s")
        base = wid * p_per_w
        pltpu.sync_copy(xs_hbm.at[pl.ds(base, p_per_w)], x_v.at[pl.ds(0, p_per_w)])
        pltpu.sync_copy(ys_hbm.at[pl.ds(base, p_per_w)], y_v.at[pl.ds(0, p_per_w)])
        pltpu.sync_copy(occ_hbm.at[pl.ds(base, p_per_w)], o_v)
        pltpu.sync_copy(pxb_hbm, a_v)
        pltpu.sync_copy(pyb_hbm, b_v)

        def cbody(i, cnt):
            off = i * _L
            xv = x_v[pl.ds(off, _L)]
            yv = y_v[pl.ds(off, _L)]
            ov = o_v[pl.ds(off, _L)]
            m = ov > 0.5
            plsc.store_compressed(x_v.at[pl.ds(cnt, _L)], xv, mask=m)
            plsc.store_compressed(y_v.at[pl.ds(cnt, _L)], yv, mask=m)
            return cnt + jnp.max(plsc.all_reduce_population_count(m))

        cnt = lax.fori_loop(0, nblk, cbody, jnp.int32(0))
        sent = jnp.full((_L,), _SENTINEL, jnp.float32)
        x_v[pl.ds(cnt, _L)] = sent
        y_v[pl.ds(cnt, _L)] = sent
        nblk_c = lax.shift_right_logical(cnt + (_L - 1), 4)

        inf16 = jnp.full((_L,), jnp.inf, jnp.float32)
        for g in range(_NG):
            pa = [a_v[pl.ds((g * _QG + j) * _L, _L)] for j in range(_QG)]
            pb = [b_v[pl.ds((g * _QG + j) * _L, _L)] for j in range(_QG)]

            def sbody(i, accs, pa=pa, pb=pb):
                off = i * _L
                xv = x_v[pl.ds(off, _L)]
                yv = y_v[pl.ds(off, _L)]
                out = []
                for j, acc in enumerate(accs):
                    dx = xv - pa[j]
                    dy = yv - pb[j]
                    out.append(jnp.minimum(acc, dx * dx + dy * dy))
                return tuple(out)

            accs = lax.fori_loop(0, nblk_c, sbody, (inf16,) * _QG)
            for j in range(_QG):
                acc_v[g * _QG + j, :] = accs[j]
        pltpu.sync_copy(acc_v, out_hbm.at[wid])

    return sc_kernel(xs, ys, occ, pxb, pyb)


def _tc_combine(partials_2d):
    def body(p_ref, cf_ref, md_ref):
        d2 = jnp.min(p_ref[...], axis=0, keepdims=True)
        md = jnp.sqrt(d2)
        md_ref[...] = md
        cf_ref[...] = md >= _HALF_DIAG

    return pl.pallas_call(
        body,
        out_shape=(
            jax.ShapeDtypeStruct((1, _T), jnp.bool_),
            jax.ShapeDtypeStruct((1, _T), jnp.float32),
        ),
    )(partials_2d)


def kernel(trajectory, occupancy, voxel_coords):
    n = occupancy.size
    xs = voxel_coords[..., 0].reshape(n)
    ys = voxel_coords[..., 1].reshape(n)
    occ = occupancy.reshape(n)

    px = trajectory[:, 0].astype(jnp.float32)
    py = trajectory[:, 1].astype(jnp.float32)
    pxb = jnp.broadcast_to(px[:, None], (_T, _L)).reshape(_T * _L)
    pyb = jnp.broadcast_to(py[:, None], (_T, _L)).reshape(_T * _L)

    partials = _sc_partial_min(xs, ys, occ, pxb, pyb)
    partials_2d = partials.transpose(0, 2, 1).reshape(_NW * _L, _T)
    cf, md = _tc_combine(partials_2d)
    return cf.reshape(_T), md.reshape(_T)
